# baseline (speedup 1.0000x reference)
import functools

import jax
import jax.numpy as jnp
from jax import lax
from jax.experimental import pallas as pl
from jax.experimental.pallas import tpu as pltpu
from jax.experimental.pallas import tpu_sc as plsc

_EPS = 0.0001
_B, _C, _H, _W = 32, 8, 256, 256
_HW = _H * _W
_M, _N = 500, 1000
_CT_FLAT = _M * _C
_CN_FLAT = _N * _C
_NC = 2


def _sin_poly(x):
    x2 = x * x
    p = 1.0 / 362880.0
    p = p * x2 - 1.0 / 5040.0
    p = p * x2 + 1.0 / 120.0
    p = p * x2 - 1.0 / 6.0
    p = p * x2 + 1.0
    return x * p


@functools.partial(
    pl.kernel,
    out_type=jax.ShapeDtypeStruct((_B * 5 * 16,), jnp.float32),
    mesh=plsc.VectorSubcoreMesh(core_axis_name="c", subcore_axis_name="s"),
    compiler_params=pltpu.CompilerParams(needs_layout_passes=False),
    scratch_types=[
        pltpu.VMEM((_M,), jnp.int32),
        pltpu.VMEM((_M,), jnp.int32),
        pltpu.VMEM((_C, _M), jnp.float32),
        pltpu.VMEM((_N,), jnp.int32),
        pltpu.VMEM((_N,), jnp.int32),
        pltpu.VMEM((_C, _N), jnp.float32),
        pltpu.VMEM((4 * _M,), jnp.int32),
        pltpu.VMEM((_CT_FLAT,), jnp.int32),
        pltpu.VMEM((_CN_FLAT,), jnp.int32),
        pltpu.VMEM((_CT_FLAT,), jnp.float32),
        pltpu.VMEM((_CN_FLAT,), jnp.float32),
        pltpu.VMEM((5 * 16,), jnp.float32),
        pltpu.SemaphoreType.DMA,
        pltpu.SemaphoreType.DMA,
        pltpu.SemaphoreType.DMA,
        pltpu.SemaphoreType.DMA,
        pltpu.SemaphoreType.DMA,
    ],
)
def _vploss(ct2cn_f, ct_ind_h, ct_mask_h, gt1_h, cn2ct_f, cn_ind_h, cn_mask_h,
            gt2_h, cci_h, out_h,
            ct_ind_v, ct_mask_v, gt1_v, cn_ind_v, cn_mask_v, gt2_v, cci_v,
            idx1_v, idx2_v, pred1_v, pred2_v, out_v, sem1, sem2, sem3, sem4,
            sem5):
    b = lax.axis_index("s") * _NC + lax.axis_index("c")

    ind1_cp = pltpu.async_copy(ct_ind_h.at[b], ct_ind_v, sem3)
    ind2_cp = pltpu.async_copy(cn_ind_h.at[b], cn_ind_v, sem4)
    aux_cps = [
        pltpu.async_copy(ct_mask_h.at[b], ct_mask_v, sem5),
        pltpu.async_copy(cn_mask_h.at[b], cn_mask_v, sem5),
        pltpu.async_copy(gt1_h.at[b], gt1_v, sem5),
        pltpu.async_copy(gt2_h.at[b], gt2_v, sem5),
        pltpu.async_copy(cci_h.at[b], cci_v, sem5),
    ]

    lanes = lax.iota(jnp.int32, 16)

    base1 = b * (_C * _HW)

    def _phys(ind):
        hi3 = jnp.left_shift(jnp.right_shift(ind, 11), 11)
        w7 = jnp.left_shift(jnp.bitwise_and(jnp.right_shift(ind, 7), 1), 10)
        hs = jnp.left_shift(jnp.bitwise_and(jnp.right_shift(ind, 8), 7), 7)
        wl = jnp.bitwise_and(ind, 127)
        return hi3 + w7 + hs + wl

    def build_cn(i, _):
        for u in range(2):
            pos = (2 * i + u) * 16 + lanes
            n = jnp.right_shift(pos, 3)
            ind = plsc.load_gather(cn_ind_v, [n])
            ch = jnp.left_shift(jnp.bitwise_and(pos, 7), 16)
            idx2_v[pl.ds((2 * i + u) * 16, 16)] = base1 + ch + _phys(ind)
        return 0

    ind2_cp.wait()
    lax.fori_loop(0, _CN_FLAT // 32, build_cn, 0)
    cp2 = pltpu.async_copy(cn2ct_f.at[idx2_v], pred2_v, sem2)

    def build_ct(i, _):
        for u in range(2):
            pos = (2 * i + u) * 16 + lanes
            m = jnp.right_shift(pos, 3)
            ind = plsc.load_gather(ct_ind_v, [m])
            ch = jnp.left_shift(jnp.bitwise_and(pos, 7), 16)
            idx1_v[pl.ds((2 * i + u) * 16, 16)] = base1 + ch + _phys(ind)
        return 0

    ind1_cp.wait()
    lax.fori_loop(0, _CT_FLAT // 32, build_ct, 0)
    cp1 = pltpu.async_copy(ct2cn_f.at[idx1_v], pred1_v, sem1)

    for cp in aux_cps:
        cp.wait()
    cp2.wait()

    zero = jnp.zeros((16,), jnp.float32)

    def cn_body(i, carry):
        s3, c3 = carry
        pos = i * 16 + lanes
        n = jnp.right_shift(pos, 3)
        c = jnp.bitwise_and(pos, 7)
        p2 = pred2_v[pl.ds(i * 16, 16)]
        g2 = plsc.load_gather(gt2_v, [c, n])
        mk = plsc.load_gather(cn_mask_v, [n])
        mf = mk.astype(jnp.float32)
        m3 = jnp.where(g2 == 0.0, mf, 1.0 - mf)
        return (s3 + jnp.abs(p2 - g2) * m3, c3 + m3)

    s3, c3 = lax.fori_loop(0, _CN_FLAT // 16, cn_body, (zero, zero))

    cp1.wait()

    def ct_body(i, carry):
        s1, s2, nct = carry
        pos = i * 16 + lanes
        m = jnp.right_shift(pos, 3)
        c = jnp.bitwise_and(pos, 7)
        p1 = pred1_v[pl.ds(i * 16, 16)]
        g1 = plsc.load_gather(gt1_v, [c, m])
        j = jnp.right_shift(pos, 1)
        cidx = plsc.load_gather(cci_v, [j])
        pofs = jnp.left_shift(cidx, 1) + jnp.bitwise_and(pos, 1)
        pg = plsc.load_gather(pred2_v, [pofs])
        gg = plsc.load_gather(
            gt2_v, [jnp.bitwise_and(pofs, 7), jnp.right_shift(pofs, 3)])
        mk = plsc.load_gather(ct_mask_v, [m])
        mf = mk.astype(jnp.float32)
        d1 = jnp.abs(p1 - g1)
        d2 = jnp.abs(pg - gg)
        delta = jnp.minimum((d1 + d2) / (jnp.abs(g1) + _EPS), 1.0)
        w = _sin_poly(1.570796 * delta)
        t = mf * w
        return (s1 + d1 * t, s2 + d2 * t, nct + mf)

    s1, s2, nct = lax.fori_loop(0, _CT_FLAT // 16, ct_body, (zero, zero, zero))

    out_v[pl.ds(0, 16)] = s1
    out_v[pl.ds(16, 16)] = s2
    out_v[pl.ds(32, 16)] = nct
    out_v[pl.ds(48, 16)] = s3
    out_v[pl.ds(64, 16)] = c3
    pltpu.sync_copy(out_v, out_h.at[pl.ds(b * 80, 80)])


def kernel(ct2cn, ct_ind, ct_mask, ct2cn_gt, cn2ct, cn_ind, cn_mask, cn2ct_gt,
           ct_cn_ind):
    def _phys_view(x):
        x5 = x.reshape(_B * _C, _H // 8, 8, _W // 128, 128)
        return jnp.transpose(x5, (0, 1, 3, 2, 4)).reshape(_B * _C * _HW)

    parts = _vploss(
        _phys_view(ct2cn),
        ct_ind,
        ct_mask,
        jnp.transpose(ct2cn_gt, (0, 2, 1)),
        _phys_view(cn2ct),
        cn_ind,
        cn_mask,
        jnp.transpose(cn2ct_gt, (0, 2, 1)),
        ct_cn_ind,
    )
    s = jnp.sum(parts.reshape(_B, 5, 16), axis=(0, 2))
    num_ct = s[2] + _EPS
    return (s[0] / num_ct, 0.5 * s[1] / num_ct, 0.2 * s[3] / (s[4] + _EPS))


# speedup vs baseline: 1.0345x; 1.0345x over previous
import functools

import jax
import jax.numpy as jnp
from jax import lax
from jax.experimental import pallas as pl
from jax.experimental.pallas import tpu as pltpu
from jax.experimental.pallas import tpu_sc as plsc

_EPS = 0.0001
_B, _C, _H, _W = 32, 8, 256, 256
_HW = _H * _W
_M, _N = 500, 1000
_CT_FLAT = _M * _C
_CN_FLAT = _N * _C
_NC = 2


def _sin_poly(x):
    x2 = x * x
    p = 1.0 / 362880.0
    p = p * x2 - 1.0 / 5040.0
    p = p * x2 + 1.0 / 120.0
    p = p * x2 - 1.0 / 6.0
    p = p * x2 + 1.0
    return x * p


@functools.partial(
    pl.kernel,
    out_type=jax.ShapeDtypeStruct((_B * 5 * 16,), jnp.float32),
    mesh=plsc.VectorSubcoreMesh(core_axis_name="c", subcore_axis_name="s"),
    compiler_params=pltpu.CompilerParams(needs_layout_passes=False),
    scratch_types=[
        pltpu.VMEM((_M,), jnp.int32),
        pltpu.VMEM((_M,), jnp.int32),
        pltpu.VMEM((_C, _M), jnp.float32),
        pltpu.VMEM((_N,), jnp.int32),
        pltpu.VMEM((_N,), jnp.int32),
        pltpu.VMEM((_C, _N), jnp.float32),
        pltpu.VMEM((4 * _M,), jnp.int32),
        pltpu.VMEM((_CT_FLAT,), jnp.int32),
        pltpu.VMEM((_CN_FLAT,), jnp.int32),
        pltpu.VMEM((_CT_FLAT,), jnp.float32),
        pltpu.VMEM((_CN_FLAT,), jnp.float32),
        pltpu.VMEM((5 * 16,), jnp.float32),
        pltpu.SemaphoreType.DMA,
        pltpu.SemaphoreType.DMA,
        pltpu.SemaphoreType.DMA,
        pltpu.SemaphoreType.DMA,
        pltpu.SemaphoreType.DMA,
    ],
)
def _vploss(ct2cn_f, ct_ind_h, ct_mask_h, gt1_h, cn2ct_f, cn_ind_h, cn_mask_h,
            gt2_h, cci_h, out_h,
            ct_ind_v, ct_mask_v, gt1_v, cn_ind_v, cn_mask_v, gt2_v, cci_v,
            idx1_v, idx2_v, pred1_v, pred2_v, out_v, sem1, sem2, sem3, sem4,
            sem5):
    b = lax.axis_index("s") * _NC + lax.axis_index("c")

    ind1_cp = pltpu.async_copy(ct_ind_h.at[b], ct_ind_v, sem3)
    ind2_cp = pltpu.async_copy(cn_ind_h.at[b], cn_ind_v, sem4)
    aux_cps = [
        pltpu.async_copy(ct_mask_h.at[b], ct_mask_v, sem5),
        pltpu.async_copy(cn_mask_h.at[b], cn_mask_v, sem5),
        pltpu.async_copy(gt1_h.at[b], gt1_v, sem5),
        pltpu.async_copy(gt2_h.at[b], gt2_v, sem5),
        pltpu.async_copy(cci_h.at[b], cci_v, sem5),
    ]

    lanes = lax.iota(jnp.int32, 16)

    base1 = b * (_C * _HW)

    def _phys(ind):
        hi3 = jnp.left_shift(jnp.right_shift(ind, 11), 11)
        w7 = jnp.left_shift(jnp.bitwise_and(jnp.right_shift(ind, 7), 1), 10)
        hs = jnp.left_shift(jnp.bitwise_and(jnp.right_shift(ind, 8), 7), 7)
        wl = jnp.bitwise_and(ind, 127)
        return hi3 + w7 + hs + wl

    def build_ct(i, _):
        for u in range(2):
            pos = (2 * i + u) * 16 + lanes
            m = jnp.right_shift(pos, 3)
            ind = plsc.load_gather(ct_ind_v, [m])
            ch = jnp.left_shift(jnp.bitwise_and(pos, 7), 16)
            idx1_v[pl.ds((2 * i + u) * 16, 16)] = base1 + ch + _phys(ind)
        return 0

    ind1_cp.wait()
    lax.fori_loop(0, _CT_FLAT // 32, build_ct, 0)
    cp1 = pltpu.async_copy(ct2cn_f.at[idx1_v], pred1_v, sem1)

    def build_cn(i, _):
        for u in range(2):
            pos = (2 * i + u) * 16 + lanes
            n = jnp.right_shift(pos, 3)
            ind = plsc.load_gather(cn_ind_v, [n])
            ch = jnp.left_shift(jnp.bitwise_and(pos, 7), 16)
            idx2_v[pl.ds((2 * i + u) * 16, 16)] = base1 + ch + _phys(ind)
        return 0

    ind2_cp.wait()
    lax.fori_loop(0, _CN_FLAT // 32, build_cn, 0)
    cp2 = pltpu.async_copy(cn2ct_f.at[idx2_v], pred2_v, sem2)

    for cp in aux_cps:
        cp.wait()
    cp2.wait()

    zero = jnp.zeros((16,), jnp.float32)

    def cn_body(i, carry):
        s3, c3 = carry
        pos = i * 16 + lanes
        n = jnp.right_shift(pos, 3)
        c = jnp.bitwise_and(pos, 7)
        p2 = pred2_v[pl.ds(i * 16, 16)]
        g2 = plsc.load_gather(gt2_v, [c, n])
        mk = plsc.load_gather(cn_mask_v, [n])
        mf = mk.astype(jnp.float32)
        m3 = jnp.where(g2 == 0.0, mf, 1.0 - mf)
        return (s3 + jnp.abs(p2 - g2) * m3, c3 + m3)

    s3, c3 = lax.fori_loop(0, _CN_FLAT // 16, cn_body, (zero, zero))

    cp1.wait()

    def ct_body(i, carry):
        s1, s2, nct = carry
        pos = i * 16 + lanes
        m = jnp.right_shift(pos, 3)
        c = jnp.bitwise_and(pos, 7)
        p1 = pred1_v[pl.ds(i * 16, 16)]
        g1 = plsc.load_gather(gt1_v, [c, m])
        j = jnp.right_shift(pos, 1)
        cidx = plsc.load_gather(cci_v, [j])
        pofs = jnp.left_shift(cidx, 1) + jnp.bitwise_and(pos, 1)
        pg = plsc.load_gather(pred2_v, [pofs])
        gg = plsc.load_gather(
            gt2_v, [jnp.bitwise_and(pofs, 7), jnp.right_shift(pofs, 3)])
        mk = plsc.load_gather(ct_mask_v, [m])
        mf = mk.astype(jnp.float32)
        d1 = jnp.abs(p1 - g1)
        d2 = jnp.abs(pg - gg)
        delta = jnp.minimum((d1 + d2) / (jnp.abs(g1) + _EPS), 1.0)
        w = _sin_poly(1.570796 * delta)
        t = mf * w
        return (s1 + d1 * t, s2 + d2 * t, nct + mf)

    s1, s2, nct = lax.fori_loop(0, _CT_FLAT // 16, ct_body, (zero, zero, zero))

    out_v[pl.ds(0, 16)] = s1
    out_v[pl.ds(16, 16)] = s2
    out_v[pl.ds(32, 16)] = nct
    out_v[pl.ds(48, 16)] = s3
    out_v[pl.ds(64, 16)] = c3
    pltpu.sync_copy(out_v, out_h.at[pl.ds(b * 80, 80)])


def kernel(ct2cn, ct_ind, ct_mask, ct2cn_gt, cn2ct, cn_ind, cn_mask, cn2ct_gt,
           ct_cn_ind):
    def _phys_view(x):
        x5 = x.reshape(_B * _C, _H // 8, 8, _W // 128, 128)
        return jnp.transpose(x5, (0, 1, 3, 2, 4)).reshape(_B * _C * _HW)

    parts = _vploss(
        _phys_view(ct2cn),
        ct_ind,
        ct_mask,
        jnp.transpose(ct2cn_gt, (0, 2, 1)),
        _phys_view(cn2ct),
        cn_ind,
        cn_mask,
        jnp.transpose(cn2ct_gt, (0, 2, 1)),
        ct_cn_ind,
    )
    s = jnp.sum(parts.reshape(_B, 5, 16), axis=(0, 2))
    num_ct = s[2] + _EPS
    return (s[0] / num_ct, 0.5 * s[1] / num_ct, 0.2 * s[3] / (s[4] + _EPS))


# speedup vs baseline: 1.0458x; 1.0110x over previous
import functools

import jax
import jax.numpy as jnp
from jax import lax
from jax.experimental import pallas as pl
from jax.experimental.pallas import tpu as pltpu
from jax.experimental.pallas import tpu_sc as plsc

_EPS = 0.0001
_B, _C, _H, _W = 32, 8, 256, 256
_HW = _H * _W
_M, _N = 500, 1000
_CT_FLAT = _M * _C
_CN_FLAT = _N * _C
_NC = 2


def _sin_poly(x):
    x2 = x * x
    p = 1.0 / 362880.0
    p = p * x2 - 1.0 / 5040.0
    p = p * x2 + 1.0 / 120.0
    p = p * x2 - 1.0 / 6.0
    p = p * x2 + 1.0
    return x * p


@functools.partial(
    pl.kernel,
    out_type=jax.ShapeDtypeStruct((_B * 5 * 16,), jnp.float32),
    mesh=plsc.VectorSubcoreMesh(core_axis_name="c", subcore_axis_name="s"),
    compiler_params=pltpu.CompilerParams(needs_layout_passes=False),
    scratch_types=[
        pltpu.VMEM((_M,), jnp.int32),
        pltpu.VMEM((_M,), jnp.int32),
        pltpu.VMEM((_C, _M), jnp.float32),
        pltpu.VMEM((_N,), jnp.int32),
        pltpu.VMEM((_N,), jnp.int32),
        pltpu.VMEM((_C, _N), jnp.float32),
        pltpu.VMEM((4 * _M,), jnp.int32),
        pltpu.VMEM((_CT_FLAT,), jnp.int32),
        pltpu.VMEM((_CN_FLAT,), jnp.int32),
        pltpu.VMEM((_CT_FLAT,), jnp.float32),
        pltpu.VMEM((_CN_FLAT,), jnp.float32),
        pltpu.VMEM((5 * 16,), jnp.float32),
        pltpu.SemaphoreType.DMA,
        pltpu.SemaphoreType.DMA,
        pltpu.SemaphoreType.DMA,
        pltpu.SemaphoreType.DMA,
        pltpu.SemaphoreType.DMA,
    ],
)
def _vploss(ct2cn_f, ct_ind_h, ct_mask_h, gt1_h, cn2ct_f, cn_ind_h, cn_mask_h,
            gt2_h, cci_h, out_h,
            ct_ind_v, ct_mask_v, gt1_v, cn_ind_v, cn_mask_v, gt2_v, cci_v,
            idx1_v, idx2_v, pred1_v, pred2_v, out_v, sem1, sem2, sem3, sem4,
            sem5):
    b = lax.axis_index("s") * _NC + lax.axis_index("c")

    ind1_cp = pltpu.async_copy(ct_ind_h.at[b], ct_ind_v, sem3)
    ind2_cp = pltpu.async_copy(cn_ind_h.at[b], cn_ind_v, sem4)
    aux_cps = [
        pltpu.async_copy(ct_mask_h.at[b], ct_mask_v, sem5),
        pltpu.async_copy(cn_mask_h.at[b], cn_mask_v, sem5),
        pltpu.async_copy(gt1_h.at[b], gt1_v, sem5),
        pltpu.async_copy(gt2_h.at[b], gt2_v, sem5),
        pltpu.async_copy(cci_h.at[b], cci_v, sem5),
    ]

    lanes = lax.iota(jnp.int32, 16)

    base1 = b * (_C * _HW)

    def _phys(ind):
        hi3 = jnp.left_shift(jnp.right_shift(ind, 11), 11)
        w7 = jnp.left_shift(jnp.bitwise_and(jnp.right_shift(ind, 7), 1), 10)
        hs = jnp.left_shift(jnp.bitwise_and(jnp.right_shift(ind, 8), 7), 7)
        wl = jnp.bitwise_and(ind, 127)
        return hi3 + w7 + hs + wl

    def build_ct(i, _):
        for u in range(2):
            pos = (2 * i + u) * 16 + lanes
            m = jnp.right_shift(pos, 3)
            ind = plsc.load_gather(ct_ind_v, [m])
            ch = jnp.left_shift(jnp.bitwise_and(pos, 7), 16)
            idx1_v[pl.ds((2 * i + u) * 16, 16)] = base1 + ch + _phys(ind)
        return 0

    ind1_cp.wait()
    lax.fori_loop(0, _CT_FLAT // 32, build_ct, 0)
    cp1 = pltpu.async_copy(ct2cn_f.at[idx1_v], pred1_v, sem1)

    def build_cn(i, _):
        for u in range(2):
            pos = (2 * i + u) * 16 + lanes
            n = jnp.right_shift(pos, 3)
            ind = plsc.load_gather(cn_ind_v, [n])
            ch = jnp.left_shift(jnp.bitwise_and(pos, 7), 16)
            idx2_v[pl.ds((2 * i + u) * 16, 16)] = base1 + ch + _phys(ind)
        return 0

    ind2_cp.wait()
    lax.fori_loop(0, _CN_FLAT // 32, build_cn, 0)
    cp2 = pltpu.async_copy(cn2ct_f.at[idx2_v], pred2_v, sem2)

    for cp in aux_cps:
        cp.wait()
    cp2.wait()

    zero = jnp.zeros((16,), jnp.float32)

    def cn_body(i, carry):
        s3, c3 = carry
        pos = i * 16 + lanes
        n = jnp.right_shift(pos, 3)
        c = jnp.bitwise_and(pos, 7)
        p2 = pred2_v[pl.ds(i * 16, 16)]
        g2 = plsc.load_gather(gt2_v, [c, n])
        mk = plsc.load_gather(cn_mask_v, [n])
        mf = mk.astype(jnp.float32)
        m3 = jnp.where(g2 == 0.0, mf, 1.0 - mf)
        return (s3 + jnp.abs(p2 - g2) * m3, c3 + m3)

    s3, c3 = lax.fori_loop(0, _CN_FLAT // 16, cn_body, (zero, zero))

    cp1.wait()

    def ct_body(i, carry):
        s1, s2, nct = carry
        for u in range(2):
            k = 2 * i + u
            pos = k * 16 + lanes
            m = jnp.right_shift(pos, 3)
            c = jnp.bitwise_and(pos, 7)
            p1 = pred1_v[pl.ds(k * 16, 16)]
            g1 = plsc.load_gather(gt1_v, [c, m])
            j = jnp.right_shift(pos, 1)
            cidx = plsc.load_gather(cci_v, [j])
            pofs = jnp.left_shift(cidx, 1) + jnp.bitwise_and(pos, 1)
            pg = plsc.load_gather(pred2_v, [pofs])
            gg = plsc.load_gather(
                gt2_v, [jnp.bitwise_and(pofs, 7), jnp.right_shift(pofs, 3)])
            mk = plsc.load_gather(ct_mask_v, [m])
            mf = mk.astype(jnp.float32)
            d1 = jnp.abs(p1 - g1)
            d2 = jnp.abs(pg - gg)
            delta = jnp.minimum((d1 + d2) / (jnp.abs(g1) + _EPS), 1.0)
            w = _sin_poly(1.570796 * delta)
            t = mf * w
            s1 = s1 + d1 * t
            s2 = s2 + d2 * t
            nct = nct + mf
        return (s1, s2, nct)

    s1, s2, nct = lax.fori_loop(0, _CT_FLAT // 32, ct_body, (zero, zero, zero))

    out_v[pl.ds(0, 16)] = s1
    out_v[pl.ds(16, 16)] = s2
    out_v[pl.ds(32, 16)] = nct
    out_v[pl.ds(48, 16)] = s3
    out_v[pl.ds(64, 16)] = c3
    pltpu.sync_copy(out_v, out_h.at[pl.ds(b * 80, 80)])


def kernel(ct2cn, ct_ind, ct_mask, ct2cn_gt, cn2ct, cn_ind, cn_mask, cn2ct_gt,
           ct_cn_ind):
    def _phys_view(x):
        x5 = x.reshape(_B * _C, _H // 8, 8, _W // 128, 128)
        return jnp.transpose(x5, (0, 1, 3, 2, 4)).reshape(_B * _C * _HW)

    parts = _vploss(
        _phys_view(ct2cn),
        ct_ind,
        ct_mask,
        jnp.transpose(ct2cn_gt, (0, 2, 1)),
        _phys_view(cn2ct),
        cn_ind,
        cn_mask,
        jnp.transpose(cn2ct_gt, (0, 2, 1)),
        ct_cn_ind,
    )
    s = jnp.sum(parts.reshape(_B, 5, 16), axis=(0, 2))
    num_ct = s[2] + _EPS
    return (s[0] / num_ct, 0.5 * s[1] / num_ct, 0.2 * s[3] / (s[4] + _EPS))


# speedup vs baseline: 1.0526x; 1.0065x over previous
import functools

import jax
import jax.numpy as jnp
from jax import lax
from jax.experimental import pallas as pl
from jax.experimental.pallas import tpu as pltpu
from jax.experimental.pallas import tpu_sc as plsc

_EPS = 0.0001
_B, _C, _H, _W = 32, 8, 256, 256
_HW = _H * _W
_M, _N = 500, 1000
_CT_FLAT = _M * _C
_CN_FLAT = _N * _C
_NC = 2


def _sin_poly(x):
    x2 = x * x
    p = 1.0 / 362880.0
    p = p * x2 - 1.0 / 5040.0
    p = p * x2 + 1.0 / 120.0
    p = p * x2 - 1.0 / 6.0
    p = p * x2 + 1.0
    return x * p


@functools.partial(
    pl.kernel,
    out_type=jax.ShapeDtypeStruct((_B * 5 * 16,), jnp.float32),
    mesh=plsc.VectorSubcoreMesh(core_axis_name="c", subcore_axis_name="s"),
    compiler_params=pltpu.CompilerParams(needs_layout_passes=False),
    scratch_types=[
        pltpu.VMEM((_M,), jnp.int32),
        pltpu.VMEM((_M,), jnp.int32),
        pltpu.VMEM((_C, _M), jnp.float32),
        pltpu.VMEM((_N,), jnp.int32),
        pltpu.VMEM((_N,), jnp.int32),
        pltpu.VMEM((_C, _N), jnp.float32),
        pltpu.VMEM((4 * _M,), jnp.int32),
        pltpu.VMEM((_CT_FLAT,), jnp.int32),
        pltpu.VMEM((_CN_FLAT,), jnp.int32),
        pltpu.VMEM((_CT_FLAT,), jnp.float32),
        pltpu.VMEM((_CN_FLAT,), jnp.float32),
        pltpu.VMEM((5 * 16,), jnp.float32),
        pltpu.SemaphoreType.DMA,
        pltpu.SemaphoreType.DMA,
        pltpu.SemaphoreType.DMA,
        pltpu.SemaphoreType.DMA,
        pltpu.SemaphoreType.DMA,
    ],
)
def _vploss(ct2cn_f, ct_ind_h, ct_mask_h, gt1_h, cn2ct_f, cn_ind_h, cn_mask_h,
            gt2_h, cci_h, out_h,
            ct_ind_v, ct_mask_v, gt1_v, cn_ind_v, cn_mask_v, gt2_v, cci_v,
            idx1_v, idx2_v, pred1_v, pred2_v, out_v, sem1, sem2, sem3, sem4,
            sem5):
    b = lax.axis_index("s") * _NC + lax.axis_index("c")

    ind1_cp = pltpu.async_copy(ct_ind_h.at[b], ct_ind_v, sem3)
    ind2_cp = pltpu.async_copy(cn_ind_h.at[b], cn_ind_v, sem4)
    aux_cps = [
        pltpu.async_copy(ct_mask_h.at[b], ct_mask_v, sem5),
        pltpu.async_copy(cn_mask_h.at[b], cn_mask_v, sem5),
        pltpu.async_copy(gt1_h.at[b], gt1_v, sem5),
        pltpu.async_copy(gt2_h.at[b], gt2_v, sem5),
        pltpu.async_copy(cci_h.at[b], cci_v, sem5),
    ]

    lanes = lax.iota(jnp.int32, 16)

    base1 = b * (_C * _HW)

    def _phys(ind):
        hi3 = jnp.left_shift(jnp.right_shift(ind, 11), 11)
        w7 = jnp.left_shift(jnp.bitwise_and(jnp.right_shift(ind, 7), 1), 10)
        hs = jnp.left_shift(jnp.bitwise_and(jnp.right_shift(ind, 8), 7), 7)
        wl = jnp.bitwise_and(ind, 127)
        return hi3 + w7 + hs + wl

    def build_ct(i, _):
        for u in range(2):
            pos = (2 * i + u) * 16 + lanes
            m = jnp.right_shift(pos, 3)
            ind = plsc.load_gather(ct_ind_v, [m])
            ch = jnp.left_shift(jnp.bitwise_and(pos, 7), 16)
            idx1_v[pl.ds((2 * i + u) * 16, 16)] = base1 + ch + _phys(ind)
        return 0

    ind1_cp.wait()
    lax.fori_loop(0, _CT_FLAT // 32, build_ct, 0)
    cp1 = pltpu.async_copy(ct2cn_f.at[idx1_v], pred1_v, sem1)

    def build_cn(i, _):
        for u in range(2):
            pos = (2 * i + u) * 16 + lanes
            n = jnp.right_shift(pos, 3)
            ind = plsc.load_gather(cn_ind_v, [n])
            ch = jnp.left_shift(jnp.bitwise_and(pos, 7), 16)
            idx2_v[pl.ds((2 * i + u) * 16, 16)] = base1 + ch + _phys(ind)
        return 0

    ind2_cp.wait()
    lax.fori_loop(0, _CN_FLAT // 32, build_cn, 0)
    half = _CN_FLAT // 2
    cp2a = pltpu.async_copy(
        cn2ct_f.at[idx2_v.at[pl.ds(0, half)]], pred2_v.at[pl.ds(0, half)],
        sem2)
    cp2b = pltpu.async_copy(
        cn2ct_f.at[idx2_v.at[pl.ds(half, half)]],
        pred2_v.at[pl.ds(half, half)], sem4)

    for cp in aux_cps:
        cp.wait()
    cp2a.wait()

    zero = jnp.zeros((16,), jnp.float32)

    def cn_body(i, carry):
        s3, c3 = carry
        pos = i * 16 + lanes
        n = jnp.right_shift(pos, 3)
        c = jnp.bitwise_and(pos, 7)
        p2 = pred2_v[pl.ds(i * 16, 16)]
        g2 = plsc.load_gather(gt2_v, [c, n])
        mk = plsc.load_gather(cn_mask_v, [n])
        mf = mk.astype(jnp.float32)
        m3 = jnp.where(g2 == 0.0, mf, 1.0 - mf)
        return (s3 + jnp.abs(p2 - g2) * m3, c3 + m3)

    s3, c3 = lax.fori_loop(0, _CN_FLAT // 32, cn_body, (zero, zero))
    cp2b.wait()
    s3, c3 = lax.fori_loop(_CN_FLAT // 32, _CN_FLAT // 16, cn_body, (s3, c3))

    cp1.wait()

    def ct_body(i, carry):
        s1, s2, nct = carry
        for u in range(2):
            k = 2 * i + u
            pos = k * 16 + lanes
            m = jnp.right_shift(pos, 3)
            c = jnp.bitwise_and(pos, 7)
            p1 = pred1_v[pl.ds(k * 16, 16)]
            g1 = plsc.load_gather(gt1_v, [c, m])
            j = jnp.right_shift(pos, 1)
            cidx = plsc.load_gather(cci_v, [j])
            pofs = jnp.left_shift(cidx, 1) + jnp.bitwise_and(pos, 1)
            pg = plsc.load_gather(pred2_v, [pofs])
            gg = plsc.load_gather(
                gt2_v, [jnp.bitwise_and(pofs, 7), jnp.right_shift(pofs, 3)])
            mk = plsc.load_gather(ct_mask_v, [m])
            mf = mk.astype(jnp.float32)
            d1 = jnp.abs(p1 - g1)
            d2 = jnp.abs(pg - gg)
            delta = jnp.minimum((d1 + d2) / (jnp.abs(g1) + _EPS), 1.0)
            w = _sin_poly(1.570796 * delta)
            t = mf * w
            s1 = s1 + d1 * t
            s2 = s2 + d2 * t
            nct = nct + mf
        return (s1, s2, nct)

    s1, s2, nct = lax.fori_loop(0, _CT_FLAT // 32, ct_body, (zero, zero, zero))

    out_v[pl.ds(0, 16)] = s1
    out_v[pl.ds(16, 16)] = s2
    out_v[pl.ds(32, 16)] = nct
    out_v[pl.ds(48, 16)] = s3
    out_v[pl.ds(64, 16)] = c3
    pltpu.sync_copy(out_v, out_h.at[pl.ds(b * 80, 80)])


def kernel(ct2cn, ct_ind, ct_mask, ct2cn_gt, cn2ct, cn_ind, cn_mask, cn2ct_gt,
           ct_cn_ind):
    def _phys_view(x):
        x5 = x.reshape(_B * _C, _H // 8, 8, _W // 128, 128)
        return jnp.transpose(x5, (0, 1, 3, 2, 4)).reshape(_B * _C * _HW)

    parts = _vploss(
        _phys_view(ct2cn),
        ct_ind,
        ct_mask,
        jnp.transpose(ct2cn_gt, (0, 2, 1)),
        _phys_view(cn2ct),
        cn_ind,
        cn_mask,
        jnp.transpose(cn2ct_gt, (0, 2, 1)),
        ct_cn_ind,
    )
    s = jnp.sum(parts.reshape(_B, 5, 16), axis=(0, 2))
    num_ct = s[2] + _EPS
    return (s[0] / num_ct, 0.5 * s[1] / num_ct, 0.2 * s[3] / (s[4] + _EPS))
